# baseline (speedup 1.0000x reference)
import functools

import jax
import jax.numpy as jnp
from jax import lax
from jax.experimental import pallas as pl
from jax.experimental.pallas import tpu as pltpu
from jax.experimental.pallas import tpu_sc as plsc

NUM_EXPERTS = 8
TOP_K = 2
D_MODEL = 1024
D_HID = 2048
AUX_COEFF = 0.01

T_TOKENS = 2048
N_PAIRS = T_TOKENS * TOP_K
BM = 256
NB = (N_PAIRS + NUM_EXPERTS * BM) // BM
P_ROWS = NB * BM
BH = 512
NHB = D_HID // BH

NW = 32
TPW = T_TOKENS // NW


def _cumsum0(a):
    n = a.shape[0]
    sh = 1
    while sh < n:
        a = a + jnp.concatenate(
            [jnp.zeros((sh, a.shape[1]), a.dtype), a[:-sh]], axis=0)
        sh *= 2
    return a


def _router_body(x_ref, gwt_ref, gb_ref, slots_ref, gates_ref, plen_ref,
                 aux_ref):
    T, E = T_TOKENS, NUM_EXPERTS
    logits = jnp.dot(x_ref[...], gwt_ref[...],
                     preferred_element_type=jnp.float32) + gb_ref[...]
    ids = jax.lax.broadcasted_iota(jnp.int32, (T, E), 1)
    m1 = jnp.max(logits, axis=1, keepdims=True)
    i1 = jnp.min(jnp.where(logits == m1, ids, E), axis=1, keepdims=True)
    neg = jnp.float32(-jnp.inf)
    logits_m = jnp.where(ids == i1, neg, logits)
    m2 = jnp.max(logits_m, axis=1, keepdims=True)
    i2 = jnp.min(jnp.where(logits_m == m2, ids, E), axis=1, keepdims=True)
    e21 = jnp.exp(m2 - m1)
    g1 = 1.0 / (1.0 + e21)
    g2 = e21 / (1.0 + e21)
    gates_ref[...] = jnp.concatenate([g1, g2], axis=1)
    lse = m1 + jnp.log(jnp.sum(jnp.exp(logits - m1), axis=1, keepdims=True))
    aux = (AUX_COEFF / E) * (-jnp.log(jnp.float32(E))
                             - jnp.mean(logits) + jnp.mean(lse))
    aux_ref[...] = jnp.reshape(aux, (1, 1))
    oh1 = (ids == i1).astype(jnp.int32)
    oh2 = (ids == i2).astype(jnp.int32)
    csum = _cumsum0(jnp.concatenate([oh1, oh2], axis=0))
    rank1 = jnp.sum(csum[:T] * oh1, axis=1, keepdims=True) - 1
    rank2 = jnp.sum(csum[T:] * oh2, axis=1, keepdims=True) - 1
    counts = csum[2 * T - 1:2 * T, :]
    plen = ((counts + (BM - 1)) // BM) * BM
    plen_ref[...] = plen
    ec = jax.lax.broadcasted_iota(jnp.int32, (E, E), 1)
    er = jax.lax.broadcasted_iota(jnp.int32, (E, E), 0)
    off = jnp.sum(jnp.where(ec < er, jnp.broadcast_to(plen, (E, E)), 0),
                  axis=1).reshape(1, E)
    off_b = jnp.broadcast_to(off, (T, E))
    slot1 = jnp.sum(oh1 * off_b, axis=1, keepdims=True) + rank1
    slot2 = jnp.sum(oh2 * off_b, axis=1, keepdims=True) + rank2
    slots_ref[...] = jnp.concatenate([slot1, slot2], axis=1)


def _router(x2d, gate_w, gate_b, interpret=False):
    T, E = T_TOKENS, NUM_EXPERTS
    slots, gates, plen, aux = pl.pallas_call(
        _router_body,
        out_shape=(jax.ShapeDtypeStruct((T, TOP_K), jnp.int32),
                   jax.ShapeDtypeStruct((T, TOP_K), jnp.float32),
                   jax.ShapeDtypeStruct((1, E), jnp.int32),
                   jax.ShapeDtypeStruct((1, 1), jnp.float32)),
        interpret=interpret,
    )(x2d, gate_w.T, gate_b.reshape(1, E))
    return slots, gates, plen, aux[0, 0]


def _sc_mesh():
    return plsc.VectorSubcoreMesh(core_axis_name="c", subcore_axis_name="s")


def _sc_wid():
    return lax.axis_index("s") * 2 + lax.axis_index("c")


def _scatter_body(x_hbm, slots_hbm, xs_hbm, idx_v, rows_v, sem):
    w = _sc_wid()
    pltpu.sync_copy(slots_hbm.at[w], idx_v)
    pltpu.sync_copy(x_hbm.at[pl.ds(w * TPW, TPW)], rows_v)
    pltpu.async_copy(rows_v, xs_hbm.at[idx_v.at[0]], sem).wait()
    pltpu.async_copy(rows_v, xs_hbm.at[idx_v.at[1]], sem).wait()


def _sc_scatter(x2d, slots_w):
    return pl.kernel(
        _scatter_body,
        out_type=jax.ShapeDtypeStruct((P_ROWS, D_MODEL), jnp.float32),
        mesh=_sc_mesh(),
        scratch_types=[
            pltpu.VMEM((TOP_K, TPW), jnp.int32),
            pltpu.VMEM((TPW, D_MODEL), jnp.float32),
            pltpu.SemaphoreType.DMA,
        ],
    )(x2d, slots_w)


def _gather_body(ys_hbm, slots_hbm, y1_hbm, y2_hbm, idx_v, rows_v, sem):
    w = _sc_wid()
    pltpu.sync_copy(slots_hbm.at[w], idx_v)
    pltpu.async_copy(ys_hbm.at[idx_v.at[0]], rows_v, sem).wait()
    pltpu.sync_copy(rows_v, y1_hbm.at[pl.ds(w * TPW, TPW)])
    pltpu.async_copy(ys_hbm.at[idx_v.at[1]], rows_v, sem).wait()
    pltpu.sync_copy(rows_v, y2_hbm.at[pl.ds(w * TPW, TPW)])


def _sc_gather(ys, slots_w):
    return pl.kernel(
        _gather_body,
        out_type=(jax.ShapeDtypeStruct((T_TOKENS, D_MODEL), jnp.float32),
                  jax.ShapeDtypeStruct((T_TOKENS, D_MODEL), jnp.float32)),
        mesh=_sc_mesh(),
        scratch_types=[
            pltpu.VMEM((TOP_K, TPW), jnp.int32),
            pltpu.VMEM((TPW, D_MODEL), jnp.float32),
            pltpu.SemaphoreType.DMA,
        ],
    )(ys, slots_w)


def _ffn_body(be_ref, nb_ref, xs_ref, w1_ref, w2_ref, out_ref):
    b = pl.program_id(0)
    h = pl.program_id(1)
    act = b < nb_ref[0]

    @pl.when(act)
    def _():
        part = jnp.dot(jax.nn.gelu(
            jnp.dot(xs_ref[...], w1_ref[0],
                    preferred_element_type=jnp.float32)),
            w2_ref[0], preferred_element_type=jnp.float32)

        @pl.when(h == 0)
        def _():
            out_ref[...] = part

        @pl.when(h != 0)
        def _():
            out_ref[...] += part


def _ffn_grouped(xs, w1, w2, block_expert, nb, interpret=False):
    D, H = D_MODEL, D_HID

    def xs_map(b, h, be, nbr):
        return (jnp.minimum(b, nbr[0] - 1), 0)

    def w1_map(b, h, be, nbr):
        bb = jnp.minimum(b, nbr[0] - 1)
        hh = jnp.where(b < nbr[0], h, NHB - 1)
        return (be[bb], 0, hh)

    def w2_map(b, h, be, nbr):
        bb = jnp.minimum(b, nbr[0] - 1)
        hh = jnp.where(b < nbr[0], h, NHB - 1)
        return (be[bb], hh, 0)

    grid_spec = pltpu.PrefetchScalarGridSpec(
        num_scalar_prefetch=2,
        grid=(NB, NHB),
        in_specs=[
            pl.BlockSpec((BM, D), xs_map),
            pl.BlockSpec((1, D, BH), w1_map),
            pl.BlockSpec((1, BH, D), w2_map),
        ],
        out_specs=pl.BlockSpec((BM, D), xs_map),
    )
    return pl.pallas_call(
        _ffn_body,
        grid_spec=grid_spec,
        out_shape=jax.ShapeDtypeStruct((P_ROWS, D), jnp.float32),
        interpret=interpret,
    )(block_expert, nb, xs, w1, w2)


def _combine_body(y1_ref, y2_ref, g_ref, out_ref):
    g = g_ref[...]
    out_ref[...] = g[:, 0:1] * y1_ref[...] + g[:, 1:2] * y2_ref[...]


def _combine(y1, y2, gates, interpret=False):
    T, D = T_TOKENS, D_MODEL
    RB = 512
    return pl.pallas_call(
        _combine_body,
        grid=(T // RB,),
        in_specs=[
            pl.BlockSpec((RB, D), lambda i: (i, 0)),
            pl.BlockSpec((RB, D), lambda i: (i, 0)),
            pl.BlockSpec((RB, TOP_K), lambda i: (i, 0)),
        ],
        out_specs=pl.BlockSpec((RB, D), lambda i: (i, 0)),
        out_shape=jax.ShapeDtypeStruct((T, D), jnp.float32),
        interpret=interpret,
    )(y1, y2, gates)


def _block_meta(plen):
    pl_row = plen.reshape(NUM_EXPERTS)
    off = jnp.cumsum(pl_row) - pl_row
    nb = jnp.sum(pl_row) // BM
    starts = jnp.arange(NB, dtype=jnp.int32) * BM
    inside = (starts[:, None] >= off[None, :]) & (
        starts[:, None] < (off + pl_row)[None, :])
    block_expert = jnp.sum(
        inside.astype(jnp.int32) * jnp.arange(NUM_EXPERTS, dtype=jnp.int32)[None, :],
        axis=1)
    return block_expert.astype(jnp.int32), nb.reshape(1).astype(jnp.int32)


def _moe(x, gate_w, gate_b, w1, w2):
    B, S, D = x.shape
    x2d = x.reshape(B * S, D)
    slots, gates, plen, aux = _router(x2d, gate_w, gate_b)
    block_expert, nb = _block_meta(plen)
    slots_w = slots.T.reshape(TOP_K, NW, TPW).transpose(1, 0, 2)
    xs = _sc_scatter(x2d, slots_w)
    ys = _ffn_grouped(xs, w1, w2, block_expert, nb)
    y1, y2 = _sc_gather(ys, slots_w)
    out = _combine(y1, y2, gates)
    return out.reshape(B, S, D), aux


@jax.jit
def kernel(x, gate_w, gate_b, w1, w2):
    return _moe(x, gate_w, gate_b, w1, w2)


# speedup vs baseline: 1.3346x; 1.3346x over previous
import functools

import jax
import jax.numpy as jnp
from jax import lax
from jax.experimental import pallas as pl
from jax.experimental.pallas import tpu as pltpu
from jax.experimental.pallas import tpu_sc as plsc

NUM_EXPERTS = 8
TOP_K = 2
D_MODEL = 1024
D_HID = 2048
AUX_COEFF = 0.01

T_TOKENS = 2048
N_PAIRS = T_TOKENS * TOP_K
BM = 256
NB = (N_PAIRS + NUM_EXPERTS * BM) // BM
P_ROWS = NB * BM
BH = 512
NHB = D_HID // BH

NW = 32
TPW = T_TOKENS // NW


def _cumsum0(a):
    n = a.shape[0]
    sh = 1
    while sh < n:
        a = a + jnp.concatenate(
            [jnp.zeros((sh, a.shape[1]), a.dtype), a[:-sh]], axis=0)
        sh *= 2
    return a


def _router_body(x_ref, gwt_ref, gb_ref, slots_ref, gates_ref, plen_ref,
                 aux_ref):
    T, E = T_TOKENS, NUM_EXPERTS
    logits = jnp.dot(x_ref[...], gwt_ref[...],
                     preferred_element_type=jnp.float32) + gb_ref[...]
    ids = jax.lax.broadcasted_iota(jnp.int32, (T, E), 1)
    m1 = jnp.max(logits, axis=1, keepdims=True)
    i1 = jnp.min(jnp.where(logits == m1, ids, E), axis=1, keepdims=True)
    neg = jnp.float32(-jnp.inf)
    logits_m = jnp.where(ids == i1, neg, logits)
    m2 = jnp.max(logits_m, axis=1, keepdims=True)
    i2 = jnp.min(jnp.where(logits_m == m2, ids, E), axis=1, keepdims=True)
    e21 = jnp.exp(m2 - m1)
    g1 = 1.0 / (1.0 + e21)
    g2 = e21 / (1.0 + e21)
    gates_ref[...] = jnp.concatenate([g1, g2], axis=1)
    lse = m1 + jnp.log(jnp.sum(jnp.exp(logits - m1), axis=1, keepdims=True))
    aux = (AUX_COEFF / E) * (-jnp.log(jnp.float32(E))
                             - jnp.mean(logits) + jnp.mean(lse))
    aux_ref[...] = jnp.reshape(aux, (1, 1))
    oh1 = (ids == i1).astype(jnp.int32)
    oh2 = (ids == i2).astype(jnp.int32)
    csum = _cumsum0(jnp.concatenate([oh1, oh2], axis=0))
    rank1 = jnp.sum(csum[:T] * oh1, axis=1, keepdims=True) - 1
    rank2 = jnp.sum(csum[T:] * oh2, axis=1, keepdims=True) - 1
    counts = csum[2 * T - 1:2 * T, :]
    plen = ((counts + (BM - 1)) // BM) * BM
    plen_ref[...] = plen
    ec = jax.lax.broadcasted_iota(jnp.int32, (E, E), 1)
    er = jax.lax.broadcasted_iota(jnp.int32, (E, E), 0)
    off = jnp.sum(jnp.where(ec < er, jnp.broadcast_to(plen, (E, E)), 0),
                  axis=1).reshape(1, E)
    off_b = jnp.broadcast_to(off, (T, E))
    slot1 = jnp.sum(oh1 * off_b, axis=1, keepdims=True) + rank1
    slot2 = jnp.sum(oh2 * off_b, axis=1, keepdims=True) + rank2
    slots_ref[...] = jnp.concatenate([slot1, slot2], axis=1)


def _router(x2d, gate_w, gate_b, interpret=False):
    T, E = T_TOKENS, NUM_EXPERTS
    slots, gates, plen, aux = pl.pallas_call(
        _router_body,
        out_shape=(jax.ShapeDtypeStruct((T, TOP_K), jnp.int32),
                   jax.ShapeDtypeStruct((T, TOP_K), jnp.float32),
                   jax.ShapeDtypeStruct((1, E), jnp.int32),
                   jax.ShapeDtypeStruct((1, 1), jnp.float32)),
        interpret=interpret,
    )(x2d, gate_w.T, gate_b.reshape(1, E))
    return slots, gates, plen, aux[0, 0]


def _sc_mesh():
    return plsc.VectorSubcoreMesh(core_axis_name="c", subcore_axis_name="s")


def _sc_wid():
    return lax.axis_index("s") * 2 + lax.axis_index("c")


def _scatter_body(x_hbm, slots_hbm, xs_hbm, idx_v, rows_v, sem):
    w = _sc_wid()
    pltpu.sync_copy(slots_hbm.at[w], idx_v)
    pltpu.sync_copy(x_hbm.at[pl.ds(w * TPW, TPW)], rows_v)
    pltpu.async_copy(rows_v, xs_hbm.at[idx_v.at[0]], sem).wait()
    pltpu.async_copy(rows_v, xs_hbm.at[idx_v.at[1]], sem).wait()


def _sc_scatter(x2d, slots_w):
    return pl.kernel(
        _scatter_body,
        out_type=jax.ShapeDtypeStruct((P_ROWS, D_MODEL), jnp.float32),
        mesh=_sc_mesh(),
        scratch_types=[
            pltpu.VMEM((TOP_K, TPW), jnp.int32),
            pltpu.VMEM((TPW, D_MODEL), jnp.float32),
            pltpu.SemaphoreType.DMA,
        ],
    )(x2d, slots_w)


def _gather_body(ys_hbm, slots_hbm, y1_hbm, y2_hbm, idx_v, rows_v, sem):
    w = _sc_wid()
    pltpu.sync_copy(slots_hbm.at[w], idx_v)
    pltpu.async_copy(ys_hbm.at[idx_v.at[0]], rows_v, sem).wait()
    pltpu.sync_copy(rows_v, y1_hbm.at[pl.ds(w * TPW, TPW)])
    pltpu.async_copy(ys_hbm.at[idx_v.at[1]], rows_v, sem).wait()
    pltpu.sync_copy(rows_v, y2_hbm.at[pl.ds(w * TPW, TPW)])


def _sc_gather(ys, slots_w):
    return pl.kernel(
        _gather_body,
        out_type=(jax.ShapeDtypeStruct((T_TOKENS, D_MODEL), jnp.float32),
                  jax.ShapeDtypeStruct((T_TOKENS, D_MODEL), jnp.float32)),
        mesh=_sc_mesh(),
        scratch_types=[
            pltpu.VMEM((TOP_K, TPW), jnp.int32),
            pltpu.VMEM((TPW, D_MODEL), jnp.float32),
            pltpu.SemaphoreType.DMA,
        ],
    )(ys, slots_w)


def _ffn_body(be_ref, nb_ref, xs_ref, w1_ref, w2_ref, out_ref):
    b = pl.program_id(0)

    @pl.when(b < nb_ref[0])
    def _():
        h = jax.nn.gelu(jnp.dot(xs_ref[...], w1_ref[0],
                                preferred_element_type=jnp.float32))
        out_ref[...] = jnp.dot(h, w2_ref[0],
                               preferred_element_type=jnp.float32)


def _ffn_grouped(xs, w1, w2, block_expert, nb, interpret=False):
    D, H = D_MODEL, D_HID

    def xs_map(b, be, nbr):
        return (jnp.minimum(b, nbr[0] - 1), 0)

    def w1_map(b, be, nbr):
        return (be[jnp.minimum(b, nbr[0] - 1)], 0, 0)

    def w2_map(b, be, nbr):
        return (be[jnp.minimum(b, nbr[0] - 1)], 0, 0)

    grid_spec = pltpu.PrefetchScalarGridSpec(
        num_scalar_prefetch=2,
        grid=(NB,),
        in_specs=[
            pl.BlockSpec((BM, D), xs_map),
            pl.BlockSpec((1, D, H), w1_map),
            pl.BlockSpec((1, H, D), w2_map),
        ],
        out_specs=pl.BlockSpec((BM, D), xs_map),
    )
    return pl.pallas_call(
        _ffn_body,
        grid_spec=grid_spec,
        out_shape=jax.ShapeDtypeStruct((P_ROWS, D), jnp.float32),
        interpret=interpret,
    )(block_expert, nb, xs, w1, w2)


def _combine_body(y1_ref, y2_ref, g_ref, out_ref):
    g = g_ref[...]
    out_ref[...] = g[:, 0:1] * y1_ref[...] + g[:, 1:2] * y2_ref[...]


def _combine(y1, y2, gates, interpret=False):
    T, D = T_TOKENS, D_MODEL
    RB = 512
    return pl.pallas_call(
        _combine_body,
        grid=(T // RB,),
        in_specs=[
            pl.BlockSpec((RB, D), lambda i: (i, 0)),
            pl.BlockSpec((RB, D), lambda i: (i, 0)),
            pl.BlockSpec((RB, TOP_K), lambda i: (i, 0)),
        ],
        out_specs=pl.BlockSpec((RB, D), lambda i: (i, 0)),
        out_shape=jax.ShapeDtypeStruct((T, D), jnp.float32),
        interpret=interpret,
    )(y1, y2, gates)


def _block_meta(plen):
    pl_row = plen.reshape(NUM_EXPERTS)
    off = jnp.cumsum(pl_row) - pl_row
    nb = jnp.sum(pl_row) // BM
    starts = jnp.arange(NB, dtype=jnp.int32) * BM
    inside = (starts[:, None] >= off[None, :]) & (
        starts[:, None] < (off + pl_row)[None, :])
    block_expert = jnp.sum(
        inside.astype(jnp.int32) * jnp.arange(NUM_EXPERTS, dtype=jnp.int32)[None, :],
        axis=1)
    return block_expert.astype(jnp.int32), nb.reshape(1).astype(jnp.int32)


def _moe(x, gate_w, gate_b, w1, w2):
    B, S, D = x.shape
    x2d = x.reshape(B * S, D)
    slots, gates, plen, aux = _router(x2d, gate_w, gate_b)
    block_expert, nb = _block_meta(plen)
    slots_w = slots.T.reshape(TOP_K, NW, TPW).transpose(1, 0, 2)
    xs = _sc_scatter(x2d, slots_w)
    ys = _ffn_grouped(xs, w1, w2, block_expert, nb)
    y1, y2 = _sc_gather(ys, slots_w)
    out = _combine(y1, y2, gates)
    return out.reshape(B, S, D), aux


@jax.jit
def kernel(x, gate_w, gate_b, w1, w2):
    return _moe(x, gate_w, gate_b, w1, w2)
